# baseline (speedup 1.0000x reference)
import functools

import jax
import jax.numpy as jnp
from jax import lax
from jax.experimental import pallas as pl
from jax.experimental.pallas import tpu as pltpu
from jax.experimental.pallas import tpu_sc as plsc

H = 128
NW = 32
LANES = 16


def _round_up(x, m):
    return -(-x // m) * m


def _make_g1(n_bonds, atoms_p, nb):
    npc = NW // 2
    apw = atoms_p // npc
    ck = 256
    ca = ck // nb
    nch = apw // ca
    idx_pw = apw * nb
    nbuf = 3
    nvisit = -(-nch // nbuf) * nbuf

    @functools.partial(
        pl.kernel,
        mesh=plsc.VectorSubcoreMesh(core_axis_name="c", subcore_axis_name="s"),
        out_type=[jax.ShapeDtypeStruct((atoms_p, H), jnp.float32),
                  jax.ShapeDtypeStruct((atoms_p, H), jnp.float32)],
        scratch_types=[
            pltpu.VMEM((idx_pw,), jnp.int32),
            pltpu.VMEM((nbuf * ck, H), jnp.float32),
            pltpu.VMEM((nbuf * ca, H), jnp.float32),
            pltpu.SemaphoreType.DMA,
            pltpu.SemaphoreType.DMA,
            pltpu.SemaphoreType.DMA,
            pltpu.SemaphoreType.DMA,
            pltpu.SemaphoreType.DMA,
            pltpu.SemaphoreType.DMA,
        ],
    )
    def g1(msg_a_hbm, a2b_a_hbm, msg_b_hbm, a2b_b_hbm, out_a_hbm, out_b_hbm,
           idx_v, rows_v, acc_v, *sems):
        semg = sems[0:nbuf]
        semo = sems[nbuf:2 * nbuf]
        sid = lax.axis_index("s")
        core = lax.axis_index("c")

        def run(msg_hbm, a2b_hbm, out_hbm):
            pltpu.sync_copy(a2b_hbm.at[pl.ds(sid * idx_pw, idx_pw)], idx_v)
            abase = sid * apw

            def rows_slot(b):
                return rows_v.at[pl.ds(b * ck, ck), :]

            def acc_slot(b):
                return acc_v.at[pl.ds(b * ca, ca), :]

            def start(c, b):
                pltpu.async_copy(
                    msg_hbm.at[idx_v.at[pl.ds(c * ck, ck)]], rows_slot(b),
                    semg[b])

            def wait_in(b):
                pltpu.make_async_copy(
                    msg_hbm.at[idx_v.at[pl.ds(0, ck)]], rows_slot(b),
                    semg[b]).wait()

            def wait_out(b):
                pltpu.make_async_copy(
                    acc_slot(b), out_hbm.at[pl.ds(abase, ca)],
                    semo[b]).wait()

            for b in range(nbuf):
                start(b, b)

            def outer(j, carry):
                cc = j * nbuf
                for b in range(nbuf):
                    c = cc + b

                    @pl.when(c < nch)
                    def _visit():
                        @pl.when(cc > 0)
                        def _drain():
                            wait_out(b)

                        wait_in(b)

                        def acc_a(a, carry2):
                            base = b * ck + a * nb
                            for g in range(H // LANES):
                                sl = pl.ds(g * LANES, LANES)
                                v = jnp.maximum(rows_v[base, sl], 0.0)
                                for k in range(1, nb):
                                    v = v + jnp.maximum(
                                        rows_v[base + k, sl], 0.0)
                                acc_v[b * ca + a, sl] = v
                            return carry2

                        lax.fori_loop(0, ca, acc_a, 0)
                        pltpu.async_copy(
                            acc_slot(b),
                            out_hbm.at[pl.ds(abase + c * ca, ca)], semo[b])

                        @pl.when(c + nbuf < nch)
                        def _next():
                            start(c + nbuf, b)
                return carry

            lax.fori_loop(0, nvisit // nbuf, outer, 0)
            for b in range(nbuf):
                wait_out(b)

        @pl.when(core == 0)
        def _enc_a():
            run(msg_a_hbm, a2b_a_hbm, out_a_hbm)

        @pl.when(core == 1)
        def _enc_b():
            run(msg_b_hbm, a2b_b_hbm, out_b_hbm)

    return g1


def _make_g2(bonds_p, atoms_p):
    npc = NW // 2
    bpw = bonds_p // npc
    nch = bpw // 128
    rows_pw = bonds_p // 128 // npc
    nbuf = 2
    nvisit = -(-nch // nbuf) * nbuf
    assert rows_pw == nch

    @functools.partial(
        pl.kernel,
        mesh=plsc.VectorSubcoreMesh(core_axis_name="c", subcore_axis_name="s"),
        out_type=[jax.ShapeDtypeStruct((bonds_p, H), jnp.float32),
                  jax.ShapeDtypeStruct((bonds_p, H), jnp.float32)],
        scratch_types=[
            pltpu.VMEM((rows_pw, 128), jnp.int32),
            pltpu.VMEM((rows_pw, 128), jnp.int32),
            pltpu.VMEM((nbuf * 128, H), jnp.float32),
            pltpu.VMEM((nbuf * 128, H), jnp.float32),
            pltpu.SemaphoreType.DMA,
            pltpu.SemaphoreType.DMA,
            pltpu.SemaphoreType.DMA,
            pltpu.SemaphoreType.DMA,
            pltpu.SemaphoreType.DMA,
            pltpu.SemaphoreType.DMA,
        ],
    )
    def g2(am_a_hbm, msg_a_hbm, am_b_hbm, msg_b_hbm,
           b2a_a_hbm, b2revb_a_hbm, b2a_b_hbm, b2revb_b_hbm,
           out_a_hbm, out_b_hbm, idxa_v, idxb_v, bufa_v, bufb_v, *sems):
        sema = sems[0:nbuf]
        semb = sems[nbuf:2 * nbuf]
        semo = sems[2 * nbuf:3 * nbuf]
        sid = lax.axis_index("s")
        core = lax.axis_index("c")

        def run(am_hbm, msg_hbm, b2a_hbm, b2revb_hbm, out_hbm):
            pltpu.sync_copy(b2a_hbm.at[pl.ds(sid * rows_pw, rows_pw)],
                            idxa_v)
            pltpu.sync_copy(b2revb_hbm.at[pl.ds(sid * rows_pw, rows_pw)],
                            idxb_v)
            bbase = sid * bpw

            def slot(ref, b):
                return ref.at[pl.ds(b * 128, 128), :]

            def start(c, b):
                pltpu.async_copy(am_hbm.at[idxa_v.at[c]], slot(bufa_v, b),
                                 sema[b])
                pltpu.async_copy(msg_hbm.at[idxb_v.at[c]], slot(bufb_v, b),
                                 semb[b])

            def wait_in(b):
                pltpu.make_async_copy(
                    am_hbm.at[idxa_v.at[0]], slot(bufa_v, b),
                    sema[b]).wait()
                pltpu.make_async_copy(
                    msg_hbm.at[idxb_v.at[0]], slot(bufb_v, b),
                    semb[b]).wait()

            def wait_out(b):
                pltpu.make_async_copy(
                    slot(bufa_v, b), out_hbm.at[pl.ds(bbase, 128)],
                    semo[b]).wait()

            for b in range(nbuf):
                start(b, b)

            def outer(j, carry):
                cc = j * nbuf
                for b in range(nbuf):
                    c = cc + b

                    @pl.when(c < nch)
                    def _visit():
                        wait_in(b)

                        def sub_r(r, carry2):
                            for g in range(H // LANES):
                                sl = pl.ds(g * LANES, LANES)
                                bufa_v[b * 128 + r, sl] = (
                                    bufa_v[b * 128 + r, sl]
                                    - jnp.maximum(
                                        bufb_v[b * 128 + r, sl], 0.0))
                            return carry2

                        lax.fori_loop(0, 128, sub_r, 0)
                        pltpu.async_copy(
                            slot(bufa_v, b),
                            out_hbm.at[pl.ds(bbase + c * 128, 128)],
                            semo[b])

                for b in range(nbuf):
                    c = cc + b

                    @pl.when(c < nch)
                    def _reissue():
                        wait_out(b)

                        @pl.when(c + nbuf < nch)
                        def _next():
                            start(c + nbuf, b)
                return carry

            lax.fori_loop(0, nvisit // nbuf, outer, 0)

        @pl.when(core == 0)
        def _enc_a():
            run(am_a_hbm, msg_a_hbm, b2a_a_hbm, b2revb_a_hbm, out_a_hbm)

        @pl.when(core == 1)
        def _enc_b():
            run(am_b_hbm, msg_b_hbm, b2a_b_hbm, b2revb_b_hbm, out_b_hbm)

    return g2


def _m1(f_bonds, W_i):
    n, fd = f_bonds.shape
    blk = 2000
    grid = n // blk

    def body(x_ref, w_ref, inp_ref):
        inp_ref[...] = jnp.dot(x_ref[...], w_ref[...],
                               preferred_element_type=jnp.float32)

    return pl.pallas_call(
        body,
        grid=(grid,),
        in_specs=[pl.BlockSpec((blk, fd), lambda i: (i, 0)),
                  pl.BlockSpec((fd, H), lambda i: (0, 0))],
        out_specs=pl.BlockSpec((blk, H), lambda i: (i, 0)),
        out_shape=jax.ShapeDtypeStruct((n, H), jnp.float32),
    )(f_bonds, W_i)


def _m3(inp, pre, W_h):
    n = inp.shape[0]
    blk = 2000
    grid = n // blk

    def body(i_ref, p_ref, w_ref, o_ref):
        o_ref[...] = i_ref[...] + jnp.dot(p_ref[...], w_ref[...],
                                          preferred_element_type=jnp.float32)

    return pl.pallas_call(
        body,
        grid=(grid,),
        in_specs=[pl.BlockSpec((blk, H), lambda i: (i, 0)),
                  pl.BlockSpec((blk, H), lambda i: (i, 0)),
                  pl.BlockSpec((H, H), lambda i: (0, 0))],
        out_specs=pl.BlockSpec((blk, H), lambda i: (i, 0)),
        out_shape=jax.ShapeDtypeStruct((n, H), jnp.float32),
    )(inp, pre, W_h)


def _m4(f_atoms_p, am_p, mol3d, W_o, b_o2d, n_mols):
    atoms_p, afd = f_atoms_p.shape
    blk = 512
    grid = atoms_p // blk

    def body(fa_ref, am_ref, id_ref, w_ref, b_ref, out_ref, cnt_ref):
        i = pl.program_id(0)

        @pl.when(i == 0)
        def _init():
            out_ref[...] = jnp.zeros_like(out_ref)
            cnt_ref[...] = jnp.zeros_like(cnt_ref)

        hid = (jnp.dot(fa_ref[...], w_ref[:afd, :],
                       preferred_element_type=jnp.float32)
               + jnp.dot(am_ref[...], w_ref[afd:, :],
                         preferred_element_type=jnp.float32)
               + b_ref[...])
        hid = jnp.maximum(hid, 0.0)
        ids = id_ref[0, 0, :]
        onehot = (ids[:, None]
                  == lax.broadcasted_iota(jnp.int32, (blk, n_mols), 1)
                  ).astype(jnp.float32)
        out_ref[...] += lax.dot_general(
            onehot, hid, (((0,), (0,)), ((), ())),
            preferred_element_type=jnp.float32)
        cnt_ref[...] = cnt_ref[...] + jnp.sum(onehot, axis=0)[:, None]

        @pl.when(i == grid - 1)
        def _fini():
            out_ref[...] = out_ref[...] / jnp.maximum(cnt_ref[...], 1.0)

    return pl.pallas_call(
        body,
        grid=(grid,),
        in_specs=[pl.BlockSpec((blk, afd), lambda i: (i, 0)),
                  pl.BlockSpec((blk, H), lambda i: (i, 0)),
                  pl.BlockSpec((1, 1, blk), lambda i: (i, 0, 0)),
                  pl.BlockSpec((afd + H, H), lambda i: (0, 0)),
                  pl.BlockSpec((1, H), lambda i: (0, 0))],
        out_specs=pl.BlockSpec((n_mols, H), lambda i: (0, 0)),
        out_shape=jax.ShapeDtypeStruct((n_mols, H), jnp.float32),
        scratch_shapes=[pltpu.VMEM((n_mols, H), jnp.float32)],
    )(f_atoms_p, am_p, mol3d, W_o, b_o2d)


def _prep(f_atoms, a2b, b2a, b2revb, mol_ids, n_bonds, n_mols):
    n_atoms, nb = a2b.shape
    atoms_p = _round_up(n_atoms, 2560)
    bonds_p = _round_up(n_bonds, 32768)

    apad = jnp.arange((atoms_p - n_atoms) * nb, dtype=jnp.int32) % n_bonds
    a2b_flat = jnp.concatenate([a2b.astype(jnp.int32).reshape(-1), apad])
    bpad = jnp.arange(bonds_p - n_bonds, dtype=jnp.int32)
    b2a2d = jnp.concatenate(
        [b2a.astype(jnp.int32), bpad % n_atoms]).reshape(-1, 128)
    b2revb2d = jnp.concatenate(
        [b2revb.astype(jnp.int32), bpad % n_bonds]).reshape(-1, 128)
    f_atoms_p = jnp.pad(f_atoms, ((0, atoms_p - n_atoms), (0, 0)))
    mol3d = jnp.pad(mol_ids.astype(jnp.int32), (0, atoms_p - n_atoms),
                    constant_values=n_mols).reshape(atoms_p // 512, 1, 512)
    return dict(a2b_flat=a2b_flat, b2a2d=b2a2d, b2revb2d=b2revb2d,
                f_atoms_p=f_atoms_p, mol3d=mol3d,
                atoms_p=atoms_p, bonds_p=bonds_p, nb=nb)


def kernel(f_atoms, f_bonds, a2b, b2a, b2revb, atom_mol_ids,
           ano_f_atoms, ano_f_bonds, ano_a2b, ano_b2a, ano_b2revb,
           ano_atom_mol_ids, W_i, W_h, W_o, b_o):
    depth = 3
    n_mols = 256
    n_bonds = f_bonds.shape[0]
    b_o2d = b_o.reshape(1, H)

    ea = _prep(f_atoms, a2b, b2a, b2revb, atom_mol_ids, n_bonds, n_mols)
    eb = _prep(ano_f_atoms, ano_a2b, ano_b2a, ano_b2revb, ano_atom_mol_ids,
               n_bonds, n_mols)

    g1 = _make_g1(n_bonds, ea['atoms_p'], ea['nb'])
    g2 = _make_g2(ea['bonds_p'], ea['atoms_p'])

    inp_a = _m1(f_bonds, W_i)
    inp_b = _m1(ano_f_bonds, W_i)
    msg_a, msg_b = inp_a, inp_b
    for _ in range(depth - 1):
        am_a, am_b = g1(msg_a, ea['a2b_flat'], msg_b, eb['a2b_flat'])
        pre_a, pre_b = g2(am_a, msg_a, am_b, msg_b,
                          ea['b2a2d'], ea['b2revb2d'],
                          eb['b2a2d'], eb['b2revb2d'])
        msg_a = _m3(inp_a, pre_a, W_h)
        msg_b = _m3(inp_b, pre_b, W_h)
    am_a, am_b = g1(msg_a, ea['a2b_flat'], msg_b, eb['a2b_flat'])
    mol_vecs = _m4(ea['f_atoms_p'], am_a, ea['mol3d'], W_o, b_o2d, n_mols)
    ano_mol_vecs = _m4(eb['f_atoms_p'], am_b, eb['mol3d'], W_o, b_o2d,
                       n_mols)
    return (mol_vecs, ano_mol_vecs)


# speedup vs baseline: 1.1864x; 1.1864x over previous
import functools

import jax
import jax.numpy as jnp
from jax import lax
from jax.experimental import pallas as pl
from jax.experimental.pallas import tpu as pltpu
from jax.experimental.pallas import tpu_sc as plsc

H = 128
NW = 32
LANES = 16


def _round_up(x, m):
    return -(-x // m) * m


def _make_g1(n_bonds, atoms_p, nb):
    apw = atoms_p // NW
    ck = 256
    ca = ck // nb
    nch = apw // ca
    idx_pw = apw * nb
    nbuf = 3
    nvisit = -(-nch // nbuf) * nbuf

    @functools.partial(
        pl.kernel,
        mesh=plsc.VectorSubcoreMesh(core_axis_name="c", subcore_axis_name="s"),
        out_type=jax.ShapeDtypeStruct((atoms_p, H), jnp.float32),
        scratch_types=[
            pltpu.VMEM((idx_pw,), jnp.int32),
            pltpu.VMEM((nbuf * ck, H), jnp.float32),
            pltpu.VMEM((nbuf * ca, H), jnp.float32),
            pltpu.SemaphoreType.DMA,
            pltpu.SemaphoreType.DMA,
            pltpu.SemaphoreType.DMA,
            pltpu.SemaphoreType.DMA,
            pltpu.SemaphoreType.DMA,
            pltpu.SemaphoreType.DMA,
        ],
    )
    def g1(msg_hbm, a2b_hbm, out_hbm, idx_v, rows_v, acc_v, *sems):
        semg = sems[0:nbuf]
        semo = sems[nbuf:2 * nbuf]
        wid = lax.axis_index("s") * 2 + lax.axis_index("c")
        pltpu.sync_copy(a2b_hbm.at[pl.ds(wid * idx_pw, idx_pw)], idx_v)
        abase = wid * apw

        def rows_slot(b):
            return rows_v.at[pl.ds(b * ck, ck), :]

        def acc_slot(b):
            return acc_v.at[pl.ds(b * ca, ca), :]

        def start(c, b):
            pltpu.async_copy(
                msg_hbm.at[idx_v.at[pl.ds(c * ck, ck)]], rows_slot(b),
                semg[b])

        def wait_in(b):
            pltpu.make_async_copy(
                msg_hbm.at[idx_v.at[pl.ds(0, ck)]], rows_slot(b),
                semg[b]).wait()

        def wait_out(b):
            pltpu.make_async_copy(
                acc_slot(b), out_hbm.at[pl.ds(abase, ca)], semo[b]).wait()

        for b in range(nbuf):
            start(b, b)

        def outer(j, carry):
            cc = j * nbuf
            for b in range(nbuf):
                c = cc + b

                @pl.when(c < nch)
                def _visit():
                    @pl.when(cc > 0)
                    def _drain():
                        wait_out(b)

                    wait_in(b)

                    def acc_a(a, carry2):
                        base = b * ck + a * nb
                        for g in range(H // LANES):
                            sl = pl.ds(g * LANES, LANES)
                            v = jnp.maximum(rows_v[base, sl], 0.0)
                            for k in range(1, nb):
                                v = v + jnp.maximum(rows_v[base + k, sl], 0.0)
                            acc_v[b * ca + a, sl] = v
                        return carry2

                    lax.fori_loop(0, ca, acc_a, 0)
                    pltpu.async_copy(
                        acc_slot(b), out_hbm.at[pl.ds(abase + c * ca, ca)],
                        semo[b])

                    @pl.when(c + nbuf < nch)
                    def _next():
                        start(c + nbuf, b)
            return carry

        lax.fori_loop(0, nvisit // nbuf, outer, 0)
        for b in range(nbuf):
            wait_out(b)

    return g1


def _make_g2(bonds_p, atoms_p):
    bpw = bonds_p // NW
    nch = bpw // 128
    rows_pw = bonds_p // 128 // NW
    nbuf = 3
    nvisit = -(-nch // nbuf) * nbuf
    assert rows_pw == nch

    @functools.partial(
        pl.kernel,
        mesh=plsc.VectorSubcoreMesh(core_axis_name="c", subcore_axis_name="s"),
        out_type=jax.ShapeDtypeStruct((bonds_p, H), jnp.float32),
        scratch_types=[
            pltpu.VMEM((rows_pw, 128), jnp.int32),
            pltpu.VMEM((rows_pw, 128), jnp.int32),
            pltpu.VMEM((nbuf * 128, H), jnp.float32),
            pltpu.VMEM((nbuf * 128, H), jnp.float32),
            pltpu.SemaphoreType.DMA,
            pltpu.SemaphoreType.DMA,
            pltpu.SemaphoreType.DMA,
            pltpu.SemaphoreType.DMA,
            pltpu.SemaphoreType.DMA,
            pltpu.SemaphoreType.DMA,
            pltpu.SemaphoreType.DMA,
            pltpu.SemaphoreType.DMA,
            pltpu.SemaphoreType.DMA,
        ],
    )
    def g2(am_hbm, msg_hbm, b2a_hbm, b2revb_hbm, out_hbm,
           idxa_v, idxb_v, bufa_v, bufb_v, *sems):
        sema = sems[0:nbuf]
        semb = sems[nbuf:2 * nbuf]
        semo = sems[2 * nbuf:3 * nbuf]
        wid = lax.axis_index("s") * 2 + lax.axis_index("c")
        pltpu.sync_copy(b2a_hbm.at[pl.ds(wid * rows_pw, rows_pw)], idxa_v)
        pltpu.sync_copy(b2revb_hbm.at[pl.ds(wid * rows_pw, rows_pw)], idxb_v)
        bbase = wid * bpw

        def slot(ref, b):
            return ref.at[pl.ds(b * 128, 128), :]

        def start(c, b):
            pltpu.async_copy(am_hbm.at[idxa_v.at[c]], slot(bufa_v, b),
                             sema[b])
            pltpu.async_copy(msg_hbm.at[idxb_v.at[c]], slot(bufb_v, b),
                             semb[b])

        def wait_in(b):
            pltpu.make_async_copy(
                am_hbm.at[idxa_v.at[0]], slot(bufa_v, b), sema[b]).wait()
            pltpu.make_async_copy(
                msg_hbm.at[idxb_v.at[0]], slot(bufb_v, b), semb[b]).wait()

        def wait_out(b):
            pltpu.make_async_copy(
                slot(bufa_v, b), out_hbm.at[pl.ds(bbase, 128)],
                semo[b]).wait()

        for b in range(nbuf):
            start(b, b)

        def outer(j, carry):
            cc = j * nbuf
            for b in range(nbuf):
                c = cc + b

                @pl.when(c < nch)
                def _visit():
                    wait_in(b)

                    def sub_r(r, carry2):
                        for g in range(H // LANES):
                            sl = pl.ds(g * LANES, LANES)
                            bufa_v[b * 128 + r, sl] = (
                                bufa_v[b * 128 + r, sl]
                                - jnp.maximum(bufb_v[b * 128 + r, sl], 0.0))
                        return carry2

                    lax.fori_loop(0, 128, sub_r, 0)
                    pltpu.async_copy(
                        slot(bufa_v, b),
                        out_hbm.at[pl.ds(bbase + c * 128, 128)], semo[b])

            for b in range(nbuf):
                c = cc + b

                @pl.when(c < nch)
                def _reissue():
                    wait_out(b)

                    @pl.when(c + nbuf < nch)
                    def _next():
                        start(c + nbuf, b)
            return carry

        lax.fori_loop(0, nvisit // nbuf, outer, 0)

    return g2


def _m1(f_bonds, W_i):
    n, fd = f_bonds.shape
    blk = 4000
    grid = n // blk

    def body(x_ref, w_ref, inp_ref):
        inp_ref[...] = jnp.dot(x_ref[...], w_ref[...],
                               preferred_element_type=jnp.float32)

    return pl.pallas_call(
        body,
        grid=(grid,),
        in_specs=[pl.BlockSpec((blk, fd), lambda i: (i, 0)),
                  pl.BlockSpec((fd, H), lambda i: (0, 0))],
        out_specs=pl.BlockSpec((blk, H), lambda i: (i, 0)),
        out_shape=jax.ShapeDtypeStruct((n, H), jnp.float32),
    )(f_bonds, W_i)


def _m3(inp, pre, W_h):
    n = inp.shape[0]
    blk = 4000
    grid = n // blk

    def body(i_ref, p_ref, w_ref, o_ref):
        o_ref[...] = i_ref[...] + jnp.dot(p_ref[...], w_ref[...],
                                          preferred_element_type=jnp.float32)

    return pl.pallas_call(
        body,
        grid=(grid,),
        in_specs=[pl.BlockSpec((blk, H), lambda i: (i, 0)),
                  pl.BlockSpec((blk, H), lambda i: (i, 0)),
                  pl.BlockSpec((H, H), lambda i: (0, 0))],
        out_specs=pl.BlockSpec((blk, H), lambda i: (i, 0)),
        out_shape=jax.ShapeDtypeStruct((n, H), jnp.float32),
    )(inp, pre, W_h)


def _m4(f_atoms_p, am_p, mol3d, W_o, b_o2d, n_mols):
    atoms_p, afd = f_atoms_p.shape
    blk = 512
    grid = atoms_p // blk

    def body(fa_ref, am_ref, id_ref, w_ref, b_ref, out_ref, cnt_ref):
        i = pl.program_id(0)

        @pl.when(i == 0)
        def _init():
            out_ref[...] = jnp.zeros_like(out_ref)
            cnt_ref[...] = jnp.zeros_like(cnt_ref)

        hid = (jnp.dot(fa_ref[...], w_ref[:afd, :],
                       preferred_element_type=jnp.float32)
               + jnp.dot(am_ref[...], w_ref[afd:, :],
                         preferred_element_type=jnp.float32)
               + b_ref[...])
        hid = jnp.maximum(hid, 0.0)
        ids = id_ref[0, 0, :]
        onehot = (ids[:, None]
                  == lax.broadcasted_iota(jnp.int32, (blk, n_mols), 1)
                  ).astype(jnp.float32)
        out_ref[...] += lax.dot_general(
            onehot, hid, (((0,), (0,)), ((), ())),
            preferred_element_type=jnp.float32)
        cnt_ref[...] = cnt_ref[...] + jnp.sum(onehot, axis=0)[:, None]

        @pl.when(i == grid - 1)
        def _fini():
            out_ref[...] = out_ref[...] / jnp.maximum(cnt_ref[...], 1.0)

    return pl.pallas_call(
        body,
        grid=(grid,),
        in_specs=[pl.BlockSpec((blk, afd), lambda i: (i, 0)),
                  pl.BlockSpec((blk, H), lambda i: (i, 0)),
                  pl.BlockSpec((1, 1, blk), lambda i: (i, 0, 0)),
                  pl.BlockSpec((afd + H, H), lambda i: (0, 0)),
                  pl.BlockSpec((1, H), lambda i: (0, 0))],
        out_specs=pl.BlockSpec((n_mols, H), lambda i: (0, 0)),
        out_shape=jax.ShapeDtypeStruct((n_mols, H), jnp.float32),
        scratch_shapes=[pltpu.VMEM((n_mols, H), jnp.float32)],
    )(f_atoms_p, am_p, mol3d, W_o, b_o2d)


def _prep(f_atoms, a2b, b2a, b2revb, mol_ids, n_bonds, n_mols):
    n_atoms, nb = a2b.shape
    atoms_p = _round_up(n_atoms, 2560)
    bonds_p = _round_up(n_bonds, 32768)

    apad = jnp.arange((atoms_p - n_atoms) * nb, dtype=jnp.int32) % n_bonds
    a2b_flat = jnp.concatenate([a2b.astype(jnp.int32).reshape(-1), apad])
    bpad = jnp.arange(bonds_p - n_bonds, dtype=jnp.int32)
    b2a2d = jnp.concatenate(
        [b2a.astype(jnp.int32), bpad % n_atoms]).reshape(-1, 128)
    b2revb2d = jnp.concatenate(
        [b2revb.astype(jnp.int32), bpad % n_bonds]).reshape(-1, 128)
    f_atoms_p = jnp.pad(f_atoms, ((0, atoms_p - n_atoms), (0, 0)))
    mol3d = jnp.pad(mol_ids.astype(jnp.int32), (0, atoms_p - n_atoms),
                    constant_values=n_mols).reshape(atoms_p // 512, 1, 512)
    return dict(a2b_flat=a2b_flat, b2a2d=b2a2d, b2revb2d=b2revb2d,
                f_atoms_p=f_atoms_p, mol3d=mol3d,
                atoms_p=atoms_p, bonds_p=bonds_p, nb=nb)


def kernel(f_atoms, f_bonds, a2b, b2a, b2revb, atom_mol_ids,
           ano_f_atoms, ano_f_bonds, ano_a2b, ano_b2a, ano_b2revb,
           ano_atom_mol_ids, W_i, W_h, W_o, b_o):
    depth = 3
    n_mols = 256
    n_bonds = f_bonds.shape[0]
    b_o2d = b_o.reshape(1, H)

    ea = _prep(f_atoms, a2b, b2a, b2revb, atom_mol_ids, n_bonds, n_mols)
    eb = _prep(ano_f_atoms, ano_a2b, ano_b2a, ano_b2revb, ano_atom_mol_ids,
               n_bonds, n_mols)

    g1 = _make_g1(n_bonds, ea['atoms_p'], ea['nb'])
    g2 = _make_g2(ea['bonds_p'], ea['atoms_p'])

    def sc_stage(e, msg):
        am = g1(msg, e['a2b_flat'])
        return g2(am, msg, e['b2a2d'], e['b2revb2d'])

    inp_a = _m1(f_bonds, W_i)
    msg_a = inp_a
    pre_a = sc_stage(ea, msg_a)
    inp_b = _m1(ano_f_bonds, W_i)
    msg_b = inp_b
    pre_b = sc_stage(eb, msg_b)
    for _ in range(depth - 2):
        msg_a = _m3(inp_a, pre_a, W_h)
        pre_a = sc_stage(ea, msg_a)
        msg_b = _m3(inp_b, pre_b, W_h)
        pre_b = sc_stage(eb, msg_b)
    msg_a = _m3(inp_a, pre_a, W_h)
    am_a = g1(msg_a, ea['a2b_flat'])
    msg_b = _m3(inp_b, pre_b, W_h)
    am_b = g1(msg_b, eb['a2b_flat'])
    mol_vecs = _m4(ea['f_atoms_p'], am_a, ea['mol3d'], W_o, b_o2d, n_mols)
    ano_mol_vecs = _m4(eb['f_atoms_p'], am_b, eb['mol3d'], W_o, b_o2d,
                       n_mols)
    return (mol_vecs, ano_mol_vecs)


# speedup vs baseline: 1.2013x; 1.0126x over previous
import functools

import jax
import jax.numpy as jnp
from jax import lax
from jax.experimental import pallas as pl
from jax.experimental.pallas import tpu as pltpu
from jax.experimental.pallas import tpu_sc as plsc

H = 128
NW = 32
LANES = 16


def _round_up(x, m):
    return -(-x // m) * m


def _make_g1(n_bonds, atoms_p, nb):
    apw = atoms_p // NW
    ck = 256
    ca = ck // nb
    nch = apw // ca
    idx_pw = apw * nb
    nbuf = 3
    nvisit = -(-nch // nbuf) * nbuf

    @functools.partial(
        pl.kernel,
        mesh=plsc.VectorSubcoreMesh(core_axis_name="c", subcore_axis_name="s"),
        out_type=jax.ShapeDtypeStruct((atoms_p, H), jnp.float32),
        scratch_types=[
            pltpu.VMEM((idx_pw,), jnp.int32),
            pltpu.VMEM((nbuf * ck, H), jnp.float32),
            pltpu.VMEM((nbuf * ca, H), jnp.float32),
            pltpu.SemaphoreType.DMA,
            pltpu.SemaphoreType.DMA,
            pltpu.SemaphoreType.DMA,
            pltpu.SemaphoreType.DMA,
            pltpu.SemaphoreType.DMA,
            pltpu.SemaphoreType.DMA,
        ],
    )
    def g1(msg_hbm, a2b_hbm, out_hbm, idx_v, rows_v, acc_v, *sems):
        semg = sems[0:nbuf]
        semo = sems[nbuf:2 * nbuf]
        wid = lax.axis_index("s") * 2 + lax.axis_index("c")
        pltpu.sync_copy(a2b_hbm.at[pl.ds(wid * idx_pw, idx_pw)], idx_v)
        abase = wid * apw

        def rows_slot(b):
            return rows_v.at[pl.ds(b * ck, ck), :]

        def acc_slot(b):
            return acc_v.at[pl.ds(b * ca, ca), :]

        def start(c, b):
            pltpu.async_copy(
                msg_hbm.at[idx_v.at[pl.ds(c * ck, ck)]], rows_slot(b),
                semg[b])

        def wait_in(b):
            pltpu.make_async_copy(
                msg_hbm.at[idx_v.at[pl.ds(0, ck)]], rows_slot(b),
                semg[b]).wait()

        def wait_out(b):
            pltpu.make_async_copy(
                acc_slot(b), out_hbm.at[pl.ds(abase, ca)], semo[b]).wait()

        for b in range(nbuf):
            start(b, b)

        def outer(j, carry):
            cc = j * nbuf
            for b in range(nbuf):
                c = cc + b

                @pl.when(c < nch)
                def _visit():
                    @pl.when(cc > 0)
                    def _drain():
                        wait_out(b)

                    wait_in(b)

                    def acc_a(a, carry2):
                        base = b * ck + a * nb
                        for g in range(H // LANES):
                            sl = pl.ds(g * LANES, LANES)
                            v = jnp.maximum(rows_v[base, sl], 0.0)
                            for k in range(1, nb):
                                v = v + jnp.maximum(rows_v[base + k, sl], 0.0)
                            acc_v[b * ca + a, sl] = v
                        return carry2

                    lax.fori_loop(0, ca, acc_a, 0)
                    pltpu.async_copy(
                        acc_slot(b), out_hbm.at[pl.ds(abase + c * ca, ca)],
                        semo[b])

                    @pl.when(c + nbuf < nch)
                    def _next():
                        start(c + nbuf, b)
            return carry

        lax.fori_loop(0, nvisit // nbuf, outer, 0)
        for b in range(nbuf):
            wait_out(b)

    return g1


def _make_g2(bonds_p, atoms_p):
    bpw = bonds_p // NW
    nch = bpw // 128
    rows_pw = bonds_p // 128 // NW
    nbuf = 3
    nvisit = -(-nch // nbuf) * nbuf
    assert rows_pw == nch

    @functools.partial(
        pl.kernel,
        mesh=plsc.VectorSubcoreMesh(core_axis_name="c", subcore_axis_name="s"),
        out_type=jax.ShapeDtypeStruct((bonds_p, H), jnp.float32),
        scratch_types=[
            pltpu.VMEM((rows_pw, 128), jnp.int32),
            pltpu.VMEM((rows_pw, 128), jnp.int32),
            pltpu.VMEM((nbuf * 128, H), jnp.float32),
            pltpu.VMEM((nbuf * 128, H), jnp.float32),
            pltpu.SemaphoreType.DMA,
            pltpu.SemaphoreType.DMA,
            pltpu.SemaphoreType.DMA,
            pltpu.SemaphoreType.DMA,
            pltpu.SemaphoreType.DMA,
            pltpu.SemaphoreType.DMA,
            pltpu.SemaphoreType.DMA,
            pltpu.SemaphoreType.DMA,
            pltpu.SemaphoreType.DMA,
        ],
    )
    def g2(am_hbm, msg_hbm, b2a_hbm, b2revb_hbm, out_hbm,
           idxa_v, idxb_v, bufa_v, bufb_v, *sems):
        sema = sems[0:nbuf]
        semb = sems[nbuf:2 * nbuf]
        semo = sems[2 * nbuf:3 * nbuf]
        wid = lax.axis_index("s") * 2 + lax.axis_index("c")
        pltpu.sync_copy(b2a_hbm.at[pl.ds(wid * rows_pw, rows_pw)], idxa_v)
        pltpu.sync_copy(b2revb_hbm.at[pl.ds(wid * rows_pw, rows_pw)], idxb_v)
        bbase = wid * bpw

        def slot(ref, b):
            return ref.at[pl.ds(b * 128, 128), :]

        def start(c, b):
            pltpu.async_copy(am_hbm.at[idxa_v.at[c]], slot(bufa_v, b),
                             sema[b])
            pltpu.async_copy(msg_hbm.at[idxb_v.at[c]], slot(bufb_v, b),
                             semb[b])

        def wait_in(b):
            pltpu.make_async_copy(
                am_hbm.at[idxa_v.at[0]], slot(bufa_v, b), sema[b]).wait()
            pltpu.make_async_copy(
                msg_hbm.at[idxb_v.at[0]], slot(bufb_v, b), semb[b]).wait()

        def wait_out(b):
            pltpu.make_async_copy(
                slot(bufa_v, b), out_hbm.at[pl.ds(bbase, 128)],
                semo[b]).wait()

        for b in range(nbuf):
            start(b, b)

        def outer(j, carry):
            cc = j * nbuf
            for b in range(nbuf):
                c = cc + b

                @pl.when(c < nch)
                def _visit():
                    wait_in(b)

                    def sub_r(r, carry2):
                        for g in range(H // LANES):
                            sl = pl.ds(g * LANES, LANES)
                            bufa_v[b * 128 + r, sl] = (
                                bufa_v[b * 128 + r, sl]
                                - jnp.maximum(bufb_v[b * 128 + r, sl], 0.0))
                        return carry2

                    lax.fori_loop(0, 128, sub_r, 0)
                    pltpu.async_copy(
                        slot(bufa_v, b),
                        out_hbm.at[pl.ds(bbase + c * 128, 128)], semo[b])

            for b in range(nbuf):
                c = cc + b

                @pl.when(c < nch)
                def _reissue():
                    wait_out(b)

                    @pl.when(c + nbuf < nch)
                    def _next():
                        start(c + nbuf, b)
            return carry

        lax.fori_loop(0, nvisit // nbuf, outer, 0)

    return g2


def _m1(f_bonds, W_i):
    n, fd = f_bonds.shape
    blk = 8000
    grid = n // blk

    def body(x_ref, w_ref, inp_ref):
        inp_ref[...] = jnp.dot(x_ref[...], w_ref[...],
                               preferred_element_type=jnp.float32)

    return pl.pallas_call(
        body,
        grid=(grid,),
        in_specs=[pl.BlockSpec((blk, fd), lambda i: (i, 0)),
                  pl.BlockSpec((fd, H), lambda i: (0, 0))],
        out_specs=pl.BlockSpec((blk, H), lambda i: (i, 0)),
        out_shape=jax.ShapeDtypeStruct((n, H), jnp.float32),
    )(f_bonds, W_i)


def _m3(inp, pre, W_h):
    n = inp.shape[0]
    blk = 8000
    grid = n // blk

    def body(i_ref, p_ref, w_ref, o_ref):
        o_ref[...] = i_ref[...] + jnp.dot(p_ref[...], w_ref[...],
                                          preferred_element_type=jnp.float32)

    return pl.pallas_call(
        body,
        grid=(grid,),
        in_specs=[pl.BlockSpec((blk, H), lambda i: (i, 0)),
                  pl.BlockSpec((blk, H), lambda i: (i, 0)),
                  pl.BlockSpec((H, H), lambda i: (0, 0))],
        out_specs=pl.BlockSpec((blk, H), lambda i: (i, 0)),
        out_shape=jax.ShapeDtypeStruct((n, H), jnp.float32),
    )(inp, pre, W_h)


def _m4(f_atoms_p, am_p, mol3d, W_o, b_o2d, n_mols):
    atoms_p, afd = f_atoms_p.shape
    blk = 2048
    grid = atoms_p // blk

    def body(fa_ref, am_ref, id_ref, w_ref, b_ref, out_ref, cnt_ref):
        i = pl.program_id(0)

        @pl.when(i == 0)
        def _init():
            out_ref[...] = jnp.zeros_like(out_ref)
            cnt_ref[...] = jnp.zeros_like(cnt_ref)

        hid = (jnp.dot(fa_ref[...], w_ref[:afd, :],
                       preferred_element_type=jnp.float32)
               + jnp.dot(am_ref[...], w_ref[afd:, :],
                         preferred_element_type=jnp.float32)
               + b_ref[...])
        hid = jnp.maximum(hid, 0.0)
        ids = id_ref[0, 0, :]
        onehot = (ids[:, None]
                  == lax.broadcasted_iota(jnp.int32, (blk, n_mols), 1)
                  ).astype(jnp.float32)
        out_ref[...] += lax.dot_general(
            onehot, hid, (((0,), (0,)), ((), ())),
            preferred_element_type=jnp.float32)
        cnt_ref[...] = cnt_ref[...] + jnp.sum(onehot, axis=0)[:, None]

        @pl.when(i == grid - 1)
        def _fini():
            out_ref[...] = out_ref[...] / jnp.maximum(cnt_ref[...], 1.0)

    return pl.pallas_call(
        body,
        grid=(grid,),
        in_specs=[pl.BlockSpec((blk, afd), lambda i: (i, 0)),
                  pl.BlockSpec((blk, H), lambda i: (i, 0)),
                  pl.BlockSpec((1, 1, blk), lambda i: (i, 0, 0)),
                  pl.BlockSpec((afd + H, H), lambda i: (0, 0)),
                  pl.BlockSpec((1, H), lambda i: (0, 0))],
        out_specs=pl.BlockSpec((n_mols, H), lambda i: (0, 0)),
        out_shape=jax.ShapeDtypeStruct((n_mols, H), jnp.float32),
        scratch_shapes=[pltpu.VMEM((n_mols, H), jnp.float32)],
    )(f_atoms_p, am_p, mol3d, W_o, b_o2d)


def _prep(f_atoms, a2b, b2a, b2revb, mol_ids, n_bonds, n_mols):
    n_atoms, nb = a2b.shape
    atoms_p = _round_up(n_atoms, 2560)
    bonds_p = _round_up(n_bonds, 32768)

    apad = jnp.arange((atoms_p - n_atoms) * nb, dtype=jnp.int32) % n_bonds
    a2b_flat = jnp.concatenate([a2b.astype(jnp.int32).reshape(-1), apad])
    bpad = jnp.arange(bonds_p - n_bonds, dtype=jnp.int32)
    b2a2d = jnp.concatenate(
        [b2a.astype(jnp.int32), bpad % n_atoms]).reshape(-1, 128)
    b2revb2d = jnp.concatenate(
        [b2revb.astype(jnp.int32), bpad % n_bonds]).reshape(-1, 128)
    f_atoms_p = jnp.pad(f_atoms, ((0, atoms_p - n_atoms), (0, 0)))
    mol3d = jnp.pad(mol_ids.astype(jnp.int32), (0, atoms_p - n_atoms),
                    constant_values=n_mols).reshape(atoms_p // 2048, 1, 2048)
    return dict(a2b_flat=a2b_flat, b2a2d=b2a2d, b2revb2d=b2revb2d,
                f_atoms_p=f_atoms_p, mol3d=mol3d,
                atoms_p=atoms_p, bonds_p=bonds_p, nb=nb)


def kernel(f_atoms, f_bonds, a2b, b2a, b2revb, atom_mol_ids,
           ano_f_atoms, ano_f_bonds, ano_a2b, ano_b2a, ano_b2revb,
           ano_atom_mol_ids, W_i, W_h, W_o, b_o):
    depth = 3
    n_mols = 256
    n_bonds = f_bonds.shape[0]
    b_o2d = b_o.reshape(1, H)

    ea = _prep(f_atoms, a2b, b2a, b2revb, atom_mol_ids, n_bonds, n_mols)
    eb = _prep(ano_f_atoms, ano_a2b, ano_b2a, ano_b2revb, ano_atom_mol_ids,
               n_bonds, n_mols)

    g1 = _make_g1(n_bonds, ea['atoms_p'], ea['nb'])
    g2 = _make_g2(ea['bonds_p'], ea['atoms_p'])

    def sc_stage(e, msg):
        am = g1(msg, e['a2b_flat'])
        return g2(am, msg, e['b2a2d'], e['b2revb2d'])

    inp_a = _m1(f_bonds, W_i)
    msg_a = inp_a
    pre_a = sc_stage(ea, msg_a)
    inp_b = _m1(ano_f_bonds, W_i)
    msg_b = inp_b
    pre_b = sc_stage(eb, msg_b)
    for _ in range(depth - 2):
        msg_a = _m3(inp_a, pre_a, W_h)
        pre_a = sc_stage(ea, msg_a)
        msg_b = _m3(inp_b, pre_b, W_h)
        pre_b = sc_stage(eb, msg_b)
    msg_a = _m3(inp_a, pre_a, W_h)
    am_a = g1(msg_a, ea['a2b_flat'])
    msg_b = _m3(inp_b, pre_b, W_h)
    am_b = g1(msg_b, eb['a2b_flat'])
    mol_vecs = _m4(ea['f_atoms_p'], am_a, ea['mol3d'], W_o, b_o2d, n_mols)
    ano_mol_vecs = _m4(eb['f_atoms_p'], am_b, eb['mol3d'], W_o, b_o2d,
                       n_mols)
    return (mol_vecs, ano_mol_vecs)
